# baseline (speedup 1.0000x reference)
import functools

import numpy as np
import jax
import jax.numpy as jnp
from jax import lax
from jax.experimental import pallas as pl
from jax.experimental.pallas import tpu as pltpu
from jax.experimental.pallas import tpu_sc as plsc

L = 16
F = 2
T = 2 ** 19
BASE_RES = 16
FINEST_RES = 512
N = 262144
PRIME = 2654435761
MASK = T - 1


def _level_res():
    b = np.exp((np.log(FINEST_RES) - np.log(BASE_RES)) / (L - 1))
    return [int(np.floor(BASE_RES * (b ** l))) for l in range(L)]


RES = _level_res()

NC, NS = 2, 16
NW = NC * NS
QPW = N // NW
C = 128
NCHUNK = QPW // C
NG = C // 16

DENSE_LEVELS = list(range(10))
STREAM_LEVELS = list(range(10, 16))
NSL = len(STREAM_LEVELS)
NROW = NSL * 4

PRIME_I32 = int(np.uint32(PRIME).view(np.int32))

_doff = {}
_rows = 0
for _l in DENSE_LEVELS:
    _doff[_l] = _rows
    _rows += (RES[_l] + 1) ** 2
NSEG = ((_rows + C - 1) // C + 3) // 4 * 4
DENSE_ROWS = NSEG * C
NSEG_P = (NSEG + 127) // 128 * 128
NSEG2 = 2 * NSEG_P


def _staging_idx():
    hidx = np.zeros((NSEG_P * C,), dtype=np.int64)
    p = 0
    for l in DENSE_LEVELS:
        s = RES[l] + 1
        cx, cy = np.meshgrid(np.arange(s), np.arange(s), indexing="ij")
        h = (cx.reshape(-1).astype(np.uint32) * np.uint32(1)) ^ (
            cy.reshape(-1).astype(np.uint32) * np.uint32(PRIME))
        hidx[p:p + s * s] = l * T + (h & np.uint32(MASK)).astype(np.int64)
        p += s * s
    lpart, h = hidx // T, hidx % T
    addr0 = lpart * (2 * T) + (h >> 7) * 256 + (h & 127)
    flat = np.empty((2 * NSEG_P * C,), dtype=np.int64)
    flat[0::2] = addr0
    flat[1::2] = addr0 + 128
    return flat.reshape(NSEG2, C).astype(np.int32)


STAGE_IDX = _staging_idx()


SEG_PER_W = NSEG2 // NW


@functools.cache
def _make_sc_stage():
    mesh = plsc.VectorSubcoreMesh(core_axis_name="c", subcore_axis_name="s",
                                  num_cores=NC, num_subcores=NS)
    return functools.partial(
        pl.kernel,
        out_type=jax.ShapeDtypeStruct((NSEG2 * C,), jnp.float32),
        mesh=mesh,
        compiler_params=pltpu.CompilerParams(needs_layout_passes=False),
        scratch_types=[
            pltpu.VMEM((SEG_PER_W, C), jnp.int32),
            pltpu.VMEM((SEG_PER_W * C,), jnp.float32),
            pltpu.SemaphoreType.DMA,
        ],
    )(_sc_stage_body)


def _sc_stage_body(tab, sidx_hbm, dense_out, sidxv, buf, sem):
    cid = lax.axis_index("c")
    sid = lax.axis_index("s")
    wid = sid * NC + cid
    s0 = wid * SEG_PER_W
    pltpu.sync_copy(sidx_hbm.at[pl.ds(s0, SEG_PER_W)], sidxv)
    cps = []
    for j in range(SEG_PER_W):
        cps.append(pltpu.async_copy(
            tab.at[sidxv.at[j]], buf.at[pl.ds(j * C, C)], sem))
    for cp in cps:
        cp.wait()
    pltpu.sync_copy(buf, dense_out.at[pl.ds(s0 * C, SEG_PER_W * C)])


SLABS = 8
SN = N // SLABS
SQPW = SN // NW
NCHUNK_S = SQPW // C


@functools.cache
def _make_sc_encode():
    mesh = plsc.VectorSubcoreMesh(core_axis_name="c", subcore_axis_name="s",
                                  num_cores=NC, num_subcores=NS)
    return functools.partial(
        pl.kernel,
        out_type=jax.ShapeDtypeStruct((2 * L, SN), jnp.float32),
        mesh=mesh,
        compiler_params=pltpu.CompilerParams(needs_layout_passes=False),
        scratch_types=[
            pltpu.VMEM((2 * DENSE_ROWS,), jnp.float32),
            pltpu.VMEM((4 * C,), jnp.float32),
            pltpu.VMEM((2 * NROW * 2 * C,), jnp.int32),
            pltpu.VMEM((2 * NROW * C,), jnp.float32),
            pltpu.VMEM((2 * NROW * 2 * C,), jnp.float32),
            pltpu.VMEM((2 * L, C), jnp.float32),
            pltpu.SemaphoreType.DMA((2,)),
        ],
    )(_sc_encode_body)


BANK = NROW * 2 * C
WBANK = NROW * C


def _sc_encode_body(tab, xT, dense_hbm, out, dense, xv, idxb, wbuf,
                    rowsb, outv, sems):
    cid = lax.axis_index("c")
    sid = lax.axis_index("s")
    wid = sid * NC + cid

    pltpu.sync_copy(dense_hbm.at[pl.ds(0, 2 * DENSE_ROWS)], dense)

    def loadx(ch, bk):
        base = wid * SQPW + ch * C
        xo = bk * (2 * C)
        pltpu.sync_copy(xT.at[0, pl.ds(base, C)], xv.at[pl.ds(xo, C)])
        pltpu.sync_copy(xT.at[1, pl.ds(base, C)], xv.at[pl.ds(xo + C, C)])

    def p1(bk):
        bo = bk * BANK
        wo = bk * WBANK
        xo = bk * (2 * C)

        def body(g, c2):
            sl = pl.ds(g * 16, 16)
            xs = xv[pl.ds(xo + g * 16, 16)]
            ys = xv[pl.ds(xo + C + g * 16, 16)]
            for si, l in enumerate(STREAM_LEVELS):
                res = float(RES[l])
                px = xs * res
                py = ys * res
                x0 = px.astype(jnp.int32)
                y0 = py.astype(jnp.int32)
                fx = px - x0.astype(jnp.float32)
                fy = py - y0.astype(jnp.float32)
                gx = 1.0 - fx
                gy = 1.0 - fy
                m0 = y0 * jnp.int32(PRIME_I32)
                m1 = m0 + jnp.int32(PRIME_I32)
                x1 = x0 + 1
                lb2 = jnp.int32(2 * l * T)
                mk = jnp.int32(MASK)
                for c4, (hx, hm, w) in enumerate((
                        (x0, m0, gx * gy), (x0, m1, gx * fy),
                        (x1, m0, fx * gy), (x1, m1, fx * fy))):
                    r = si * 4 + c4
                    h3 = (hx ^ hm) & mk
                    hh = ((h3 + h3) - (h3 & 127)) | lb2
                    o = r * 2 * C
                    idxb[pl.ds(bo + o + g * 16, 16)] = hh
                    idxb[pl.ds(bo + o + C + g * 16, 16)] = hh + 128
                    wbuf[pl.ds(wo + r * C + g * 16, 16)] = w
            return c2

        lax.fori_loop(0, NG, body, 0)

    def fire(bk):
        bo = bk * BANK
        for k in range(NROW * 2):
            pltpu.async_copy(
                tab.at[idxb.at[pl.ds(bo + k * C, C)]],
                rowsb.at[pl.ds(bo + k * C, C)], sems.at[bk])

    def drain(bk):
        bo = bk * BANK
        pltpu.make_async_copy(
            tab.at[pl.ds(0, BANK)],
            rowsb.at[pl.ds(bo, BANK)], sems.at[bk]).wait()

    def p2(bk):
        xo = bk * (2 * C)

        def body(g, c2):
            sl = pl.ds(g * 16, 16)
            xs = xv[pl.ds(xo + g * 16, 16)]
            ys = xv[pl.ds(xo + C + g * 16, 16)]
            for l in DENSE_LEVELS:
                res = float(RES[l])
                s = RES[l] + 1
                px = xs * res
                py = ys * res
                x0 = px.astype(jnp.int32)
                y0 = py.astype(jnp.int32)
                fx = px - x0.astype(jnp.float32)
                fy = py - y0.astype(jnp.float32)
                gx = 1.0 - fx
                gy = 1.0 - fy
                d00 = x0 * jnp.int32(2 * s) + (y0 + y0) + jnp.int32(2 * _doff[l])
                d01 = d00 + 2
                d10 = d00 + jnp.int32(2 * s)
                d11 = d10 + 2
                a0 = jnp.zeros((16,), jnp.float32)
                a1 = jnp.zeros((16,), jnp.float32)
                for d, w in ((d00, gx * gy), (d01, gx * fy),
                             (d10, fx * gy), (d11, fx * fy)):
                    a0 = a0 + plsc.load_gather(dense, [d]) * w
                    a1 = a1 + plsc.load_gather(dense, [d + 1]) * w
                outv[2 * l, sl] = a0
                outv[2 * l + 1, sl] = a1
            return c2

        lax.fori_loop(0, NG, body, 0)

    def p3(bk):
        bo = bk * BANK
        wo = bk * WBANK

        def body(g, c2):
            sl = pl.ds(g * 16, 16)
            for si, l in enumerate(STREAM_LEVELS):
                a0 = jnp.zeros((16,), jnp.float32)
                a1 = jnp.zeros((16,), jnp.float32)
                for c4 in range(4):
                    r = si * 4 + c4
                    o = r * 2 * C
                    w = wbuf[pl.ds(wo + r * C + g * 16, 16)]
                    a0 = a0 + rowsb[pl.ds(bo + o + g * 16, 16)] * w
                    a1 = a1 + rowsb[pl.ds(bo + o + C + g * 16, 16)] * w
                outv[2 * l, sl] = a0
                outv[2 * l + 1, sl] = a1
            return c2

        lax.fori_loop(0, NG, body, 0)

    loadx(0, 0)
    p1(0)
    fire(0)

    def chunk(ch, carry):
        par = lax.rem(ch, 2)
        npar = 1 - par

        @pl.when(ch + 1 < NCHUNK_S)
        def _():
            loadx(ch + 1, npar)
            p1(npar)
            fire(npar)

        p2(par)
        drain(par)
        p3(par)
        pltpu.sync_copy(outv, out.at[:, pl.ds(wid * SQPW + ch * C, C)])
        return carry

    lax.fori_loop(0, NCHUNK_S, chunk, 0)


BN = 512


def _mlp_body(e_ref, w1t_ref, b1_ref, w2t_ref, b2_ref, w3_ref, b3_ref,
              o_ref):
    e = e_ref[...]
    h1 = jax.lax.dot_general(w1t_ref[...], e, (((1,), (0,)), ((), ())),
                             preferred_element_type=jnp.float32)
    h1 = jnp.maximum(h1 + b1_ref[...], 0.0)
    h2 = jax.lax.dot_general(w2t_ref[...], h1, (((1,), (0,)), ((), ())),
                             preferred_element_type=jnp.float32)
    h2 = jnp.maximum(h2 + b2_ref[...], 0.0)
    o_ref[...] = jnp.sum(h2 * w3_ref[...], axis=0, keepdims=True) + b3_ref[...]


def _mlp(enc, w1t, b1c, w2t, b2c, w3, b3c):
    return pl.pallas_call(
        _mlp_body,
        grid=(SN // BN,),
        in_specs=[
            pl.BlockSpec((2 * L, BN), lambda i: (0, i)),
            pl.BlockSpec((128, 2 * L), lambda i: (0, 0)),
            pl.BlockSpec((128, 1), lambda i: (0, 0)),
            pl.BlockSpec((128, 128), lambda i: (0, 0)),
            pl.BlockSpec((128, 1), lambda i: (0, 0)),
            pl.BlockSpec((128, 1), lambda i: (0, 0)),
            pl.BlockSpec((1, 1), lambda i: (0, 0)),
        ],
        out_specs=pl.BlockSpec((1, BN), lambda i: (0, i)),
        out_shape=jax.ShapeDtypeStruct((1, SN), jnp.float32),
    )(enc, w1t, b1c, w2t, b2c, w3, b3c)


def kernel(x, tables, W1, b1, W2, b2, W3, b3):
    xT = x.T
    tab = tables.reshape(L, T // 128, 128, F).transpose(0, 1, 3, 2)
    tab = tab.reshape(L * T * F)
    dense_arr = _make_sc_stage()(tab, jnp.asarray(STAGE_IDX))
    encode = _make_sc_encode()
    w1t, b1c = W1.T, b1.reshape(128, 1)
    w2t, b2c = W2.T, b2.reshape(128, 1)
    b3c = b3.reshape(1, 1)
    outs = []
    for s in range(SLABS):
        enc = encode(tab, xT[:, s * SN:(s + 1) * SN], dense_arr)
        outs.append(_mlp(enc, w1t, b1c, w2t, b2c, W3, b3c))
    return jnp.concatenate(outs, axis=1).reshape(N, 1)


# speedup vs baseline: 1.0028x; 1.0028x over previous
import functools

import numpy as np
import jax
import jax.numpy as jnp
from jax import lax
from jax.experimental import pallas as pl
from jax.experimental.pallas import tpu as pltpu
from jax.experimental.pallas import tpu_sc as plsc

L = 16
F = 2
T = 2 ** 19
BASE_RES = 16
FINEST_RES = 512
N = 262144
PRIME = 2654435761
MASK = T - 1


def _level_res():
    b = np.exp((np.log(FINEST_RES) - np.log(BASE_RES)) / (L - 1))
    return [int(np.floor(BASE_RES * (b ** l))) for l in range(L)]


RES = _level_res()

NC, NS = 2, 16
NW = NC * NS
QPW = N // NW
C = 128
NCHUNK = QPW // C
NG = C // 16

DENSE_LEVELS = list(range(10))
STREAM_LEVELS = list(range(10, 16))
NSL = len(STREAM_LEVELS)
NROW = NSL * 4

PRIME_I32 = int(np.uint32(PRIME).view(np.int32))

_doff = {}
_rows = 0
for _l in DENSE_LEVELS:
    _doff[_l] = _rows
    _rows += (RES[_l] + 1) ** 2
NSEG = ((_rows + C - 1) // C + 3) // 4 * 4
DENSE_ROWS = NSEG * C
NSEG_P = (NSEG + 127) // 128 * 128
NSEG2 = 2 * NSEG_P


def _staging_idx():
    hidx = np.zeros((NSEG_P * C,), dtype=np.int64)
    p = 0
    for l in DENSE_LEVELS:
        s = RES[l] + 1
        cx, cy = np.meshgrid(np.arange(s), np.arange(s), indexing="ij")
        h = (cx.reshape(-1).astype(np.uint32) * np.uint32(1)) ^ (
            cy.reshape(-1).astype(np.uint32) * np.uint32(PRIME))
        hidx[p:p + s * s] = l * T + (h & np.uint32(MASK)).astype(np.int64)
        p += s * s
    lpart, h = hidx // T, hidx % T
    addr0 = lpart * (2 * T) + (h >> 7) * 256 + (h & 127)
    flat = np.empty((2 * NSEG_P * C,), dtype=np.int64)
    flat[0::2] = addr0
    flat[1::2] = addr0 + 128
    return flat.reshape(NSEG2, C).astype(np.int32)


STAGE_IDX = _staging_idx()


SEG_PER_W = NSEG2 // NW


@functools.cache
def _make_sc_stage():
    mesh = plsc.VectorSubcoreMesh(core_axis_name="c", subcore_axis_name="s",
                                  num_cores=NC, num_subcores=NS)
    return functools.partial(
        pl.kernel,
        out_type=jax.ShapeDtypeStruct((NSEG2 * C,), jnp.float32),
        mesh=mesh,
        compiler_params=pltpu.CompilerParams(needs_layout_passes=False),
        scratch_types=[
            pltpu.VMEM((SEG_PER_W, C), jnp.int32),
            pltpu.VMEM((SEG_PER_W * C,), jnp.float32),
            pltpu.SemaphoreType.DMA,
        ],
    )(_sc_stage_body)


def _sc_stage_body(tab, sidx_hbm, dense_out, sidxv, buf, sem):
    cid = lax.axis_index("c")
    sid = lax.axis_index("s")
    wid = sid * NC + cid
    s0 = wid * SEG_PER_W
    pltpu.sync_copy(sidx_hbm.at[pl.ds(s0, SEG_PER_W)], sidxv)
    cps = []
    for j in range(SEG_PER_W):
        cps.append(pltpu.async_copy(
            tab.at[sidxv.at[j]], buf.at[pl.ds(j * C, C)], sem))
    for cp in cps:
        cp.wait()
    pltpu.sync_copy(buf, dense_out.at[pl.ds(s0 * C, SEG_PER_W * C)])


SLABS = 2
SN = N // SLABS
SQPW = SN // NW
NCHUNK_S = SQPW // C


@functools.cache
def _make_sc_encode():
    mesh = plsc.VectorSubcoreMesh(core_axis_name="c", subcore_axis_name="s",
                                  num_cores=NC, num_subcores=NS)
    return functools.partial(
        pl.kernel,
        out_type=jax.ShapeDtypeStruct((2 * L, SN), jnp.float32),
        mesh=mesh,
        compiler_params=pltpu.CompilerParams(needs_layout_passes=False),
        scratch_types=[
            pltpu.VMEM((2 * DENSE_ROWS,), jnp.float32),
            pltpu.VMEM((4 * C,), jnp.float32),
            pltpu.VMEM((2 * NROW * 2 * C,), jnp.int32),
            pltpu.VMEM((2 * NROW * C,), jnp.float32),
            pltpu.VMEM((2 * NROW * 2 * C,), jnp.float32),
            pltpu.VMEM((2 * L, C), jnp.float32),
            pltpu.SemaphoreType.DMA((2,)),
        ],
    )(_sc_encode_body)


BANK = NROW * 2 * C
WBANK = NROW * C


def _sc_encode_body(tab, xT, dense_hbm, out, dense, xv, idxb, wbuf,
                    rowsb, outv, sems):
    cid = lax.axis_index("c")
    sid = lax.axis_index("s")
    wid = sid * NC + cid

    pltpu.sync_copy(dense_hbm.at[pl.ds(0, 2 * DENSE_ROWS)], dense)

    def loadx(ch, bk):
        base = wid * SQPW + ch * C
        xo = bk * (2 * C)
        pltpu.sync_copy(xT.at[0, pl.ds(base, C)], xv.at[pl.ds(xo, C)])
        pltpu.sync_copy(xT.at[1, pl.ds(base, C)], xv.at[pl.ds(xo + C, C)])

    def p1(bk):
        bo = bk * BANK
        wo = bk * WBANK
        xo = bk * (2 * C)

        def body(g, c2):
            sl = pl.ds(g * 16, 16)
            xs = xv[pl.ds(xo + g * 16, 16)]
            ys = xv[pl.ds(xo + C + g * 16, 16)]
            for si, l in enumerate(STREAM_LEVELS):
                res = float(RES[l])
                px = xs * res
                py = ys * res
                x0 = px.astype(jnp.int32)
                y0 = py.astype(jnp.int32)
                fx = px - x0.astype(jnp.float32)
                fy = py - y0.astype(jnp.float32)
                gx = 1.0 - fx
                gy = 1.0 - fy
                m0 = y0 * jnp.int32(PRIME_I32)
                m1 = m0 + jnp.int32(PRIME_I32)
                x1 = x0 + 1
                lb2 = jnp.int32(2 * l * T)
                mk = jnp.int32(MASK)
                for c4, (hx, hm, w) in enumerate((
                        (x0, m0, gx * gy), (x0, m1, gx * fy),
                        (x1, m0, fx * gy), (x1, m1, fx * fy))):
                    r = si * 4 + c4
                    h3 = (hx ^ hm) & mk
                    hh = ((h3 + h3) - (h3 & 127)) | lb2
                    o = r * 2 * C
                    idxb[pl.ds(bo + o + g * 16, 16)] = hh
                    idxb[pl.ds(bo + o + C + g * 16, 16)] = hh + 128
                    wbuf[pl.ds(wo + r * C + g * 16, 16)] = w
            return c2

        lax.fori_loop(0, NG, body, 0)

    def fire(bk):
        bo = bk * BANK
        for k in range(NROW * 2):
            pltpu.async_copy(
                tab.at[idxb.at[pl.ds(bo + k * C, C)]],
                rowsb.at[pl.ds(bo + k * C, C)], sems.at[bk])

    def drain(bk):
        bo = bk * BANK
        pltpu.make_async_copy(
            tab.at[pl.ds(0, BANK)],
            rowsb.at[pl.ds(bo, BANK)], sems.at[bk]).wait()

    def p2(bk):
        xo = bk * (2 * C)

        def body(g, c2):
            sl = pl.ds(g * 16, 16)
            xs = xv[pl.ds(xo + g * 16, 16)]
            ys = xv[pl.ds(xo + C + g * 16, 16)]
            for l in DENSE_LEVELS:
                res = float(RES[l])
                s = RES[l] + 1
                px = xs * res
                py = ys * res
                x0 = px.astype(jnp.int32)
                y0 = py.astype(jnp.int32)
                fx = px - x0.astype(jnp.float32)
                fy = py - y0.astype(jnp.float32)
                gx = 1.0 - fx
                gy = 1.0 - fy
                d00 = x0 * jnp.int32(2 * s) + (y0 + y0) + jnp.int32(2 * _doff[l])
                d01 = d00 + 2
                d10 = d00 + jnp.int32(2 * s)
                d11 = d10 + 2
                a0 = jnp.zeros((16,), jnp.float32)
                a1 = jnp.zeros((16,), jnp.float32)
                for d, w in ((d00, gx * gy), (d01, gx * fy),
                             (d10, fx * gy), (d11, fx * fy)):
                    a0 = a0 + plsc.load_gather(dense, [d]) * w
                    a1 = a1 + plsc.load_gather(dense, [d + 1]) * w
                outv[2 * l, sl] = a0
                outv[2 * l + 1, sl] = a1
            return c2

        lax.fori_loop(0, NG, body, 0)

    def p3(bk):
        bo = bk * BANK
        wo = bk * WBANK

        def body(g, c2):
            sl = pl.ds(g * 16, 16)
            for si, l in enumerate(STREAM_LEVELS):
                a0 = jnp.zeros((16,), jnp.float32)
                a1 = jnp.zeros((16,), jnp.float32)
                for c4 in range(4):
                    r = si * 4 + c4
                    o = r * 2 * C
                    w = wbuf[pl.ds(wo + r * C + g * 16, 16)]
                    a0 = a0 + rowsb[pl.ds(bo + o + g * 16, 16)] * w
                    a1 = a1 + rowsb[pl.ds(bo + o + C + g * 16, 16)] * w
                outv[2 * l, sl] = a0
                outv[2 * l + 1, sl] = a1
            return c2

        lax.fori_loop(0, NG, body, 0)

    loadx(0, 0)
    p1(0)
    fire(0)

    def chunk(ch, carry):
        par = lax.rem(ch, 2)
        npar = 1 - par

        @pl.when(ch + 1 < NCHUNK_S)
        def _():
            loadx(ch + 1, npar)
            p1(npar)
            fire(npar)

        p2(par)
        drain(par)
        p3(par)
        pltpu.sync_copy(outv, out.at[:, pl.ds(wid * SQPW + ch * C, C)])
        return carry

    lax.fori_loop(0, NCHUNK_S, chunk, 0)


BN = 512


def _mlp_body(e_ref, w1t_ref, b1_ref, w2t_ref, b2_ref, w3_ref, b3_ref,
              o_ref):
    e = e_ref[...]
    h1 = jax.lax.dot_general(w1t_ref[...], e, (((1,), (0,)), ((), ())),
                             preferred_element_type=jnp.float32)
    h1 = jnp.maximum(h1 + b1_ref[...], 0.0)
    h2 = jax.lax.dot_general(w2t_ref[...], h1, (((1,), (0,)), ((), ())),
                             preferred_element_type=jnp.float32)
    h2 = jnp.maximum(h2 + b2_ref[...], 0.0)
    o_ref[...] = jnp.sum(h2 * w3_ref[...], axis=0, keepdims=True) + b3_ref[...]


def _mlp(enc, w1t, b1c, w2t, b2c, w3, b3c):
    return pl.pallas_call(
        _mlp_body,
        grid=(SN // BN,),
        in_specs=[
            pl.BlockSpec((2 * L, BN), lambda i: (0, i)),
            pl.BlockSpec((128, 2 * L), lambda i: (0, 0)),
            pl.BlockSpec((128, 1), lambda i: (0, 0)),
            pl.BlockSpec((128, 128), lambda i: (0, 0)),
            pl.BlockSpec((128, 1), lambda i: (0, 0)),
            pl.BlockSpec((128, 1), lambda i: (0, 0)),
            pl.BlockSpec((1, 1), lambda i: (0, 0)),
        ],
        out_specs=pl.BlockSpec((1, BN), lambda i: (0, i)),
        out_shape=jax.ShapeDtypeStruct((1, SN), jnp.float32),
    )(enc, w1t, b1c, w2t, b2c, w3, b3c)


def kernel(x, tables, W1, b1, W2, b2, W3, b3):
    xT = x.T
    tab = tables.reshape(L, T // 128, 128, F).transpose(0, 1, 3, 2)
    tab = tab.reshape(L * T * F)
    dense_arr = _make_sc_stage()(tab, jnp.asarray(STAGE_IDX))
    encode = _make_sc_encode()
    w1t, b1c = W1.T, b1.reshape(128, 1)
    w2t, b2c = W2.T, b2.reshape(128, 1)
    b3c = b3.reshape(1, 1)
    outs = []
    for s in range(SLABS):
        enc = encode(tab, xT[:, s * SN:(s + 1) * SN], dense_arr)
        outs.append(_mlp(enc, w1t, b1c, w2t, b2c, W3, b3c))
    return jnp.concatenate(outs, axis=1).reshape(N, 1)


# speedup vs baseline: 1.0515x; 1.0486x over previous
import functools

import numpy as np
import jax
import jax.numpy as jnp
from jax import lax
from jax.experimental import pallas as pl
from jax.experimental.pallas import tpu as pltpu
from jax.experimental.pallas import tpu_sc as plsc

L = 16
F = 2
T = 2 ** 19
BASE_RES = 16
FINEST_RES = 512
N = 262144
PRIME = 2654435761
MASK = T - 1


def _level_res():
    b = np.exp((np.log(FINEST_RES) - np.log(BASE_RES)) / (L - 1))
    return [int(np.floor(BASE_RES * (b ** l))) for l in range(L)]


RES = _level_res()

NC, NS = 2, 16
NW = NC * NS
QPW = N // NW
C = 128
NCHUNK = QPW // C
NG = C // 16

DENSE_LEVELS = list(range(10))
STREAM_LEVELS = list(range(10, 16))
NSL = len(STREAM_LEVELS)
NROW = NSL * 4

PRIME_I32 = int(np.uint32(PRIME).view(np.int32))

_doff = {}
_rows = 0
for _l in DENSE_LEVELS:
    _doff[_l] = _rows
    _rows += (RES[_l] + 1) ** 2
NSEG = ((_rows + C - 1) // C + 3) // 4 * 4
DENSE_ROWS = NSEG * C
NSEG_P = (NSEG + 127) // 128 * 128
NSEG2 = 2 * NSEG_P


def _staging_idx():
    hidx = np.zeros((NSEG_P * C,), dtype=np.int64)
    p = 0
    for l in DENSE_LEVELS:
        s = RES[l] + 1
        cx, cy = np.meshgrid(np.arange(s), np.arange(s), indexing="ij")
        h = (cx.reshape(-1).astype(np.uint32) * np.uint32(1)) ^ (
            cy.reshape(-1).astype(np.uint32) * np.uint32(PRIME))
        hidx[p:p + s * s] = l * T + (h & np.uint32(MASK)).astype(np.int64)
        p += s * s
    lpart, h = hidx // T, hidx % T
    addr0 = lpart * (2 * T) + (h >> 7) * 256 + (h & 127)
    flat = np.empty((2 * NSEG_P * C,), dtype=np.int64)
    flat[0::2] = addr0
    flat[1::2] = addr0 + 128
    return flat.reshape(NSEG2, C).astype(np.int32)


STAGE_IDX = _staging_idx()


SEG_PER_W = NSEG2 // NW


@functools.cache
def _make_sc_stage():
    mesh = plsc.VectorSubcoreMesh(core_axis_name="c", subcore_axis_name="s",
                                  num_cores=NC, num_subcores=NS)
    return functools.partial(
        pl.kernel,
        out_type=jax.ShapeDtypeStruct((NSEG2 * C,), jnp.float32),
        mesh=mesh,
        compiler_params=pltpu.CompilerParams(needs_layout_passes=False),
        scratch_types=[
            pltpu.VMEM((SEG_PER_W, C), jnp.int32),
            pltpu.VMEM((SEG_PER_W * C,), jnp.float32),
            pltpu.SemaphoreType.DMA,
        ],
    )(_sc_stage_body)


def _sc_stage_body(tab, sidx_hbm, dense_out, sidxv, buf, sem):
    cid = lax.axis_index("c")
    sid = lax.axis_index("s")
    wid = sid * NC + cid
    s0 = wid * SEG_PER_W
    pltpu.sync_copy(sidx_hbm.at[pl.ds(s0, SEG_PER_W)], sidxv)
    cps = []
    for j in range(SEG_PER_W):
        cps.append(pltpu.async_copy(
            tab.at[sidxv.at[j]], buf.at[pl.ds(j * C, C)], sem))
    for cp in cps:
        cp.wait()
    pltpu.sync_copy(buf, dense_out.at[pl.ds(s0 * C, SEG_PER_W * C)])


SLABS = 4
SN = N // SLABS
SQPW = SN // NW
NCHUNK_S = SQPW // C


@functools.cache
def _make_sc_encode():
    mesh = plsc.VectorSubcoreMesh(core_axis_name="c", subcore_axis_name="s",
                                  num_cores=NC, num_subcores=NS)
    return functools.partial(
        pl.kernel,
        out_type=jax.ShapeDtypeStruct((2 * L, SN), jnp.float32),
        mesh=mesh,
        compiler_params=pltpu.CompilerParams(needs_layout_passes=False),
        scratch_types=[
            pltpu.VMEM((2 * DENSE_ROWS,), jnp.float32),
            pltpu.VMEM((4 * C,), jnp.float32),
            pltpu.VMEM((2 * NROW * 2 * C,), jnp.int32),
            pltpu.VMEM((2 * NROW * C,), jnp.float32),
            pltpu.VMEM((2 * NROW * 2 * C,), jnp.float32),
            pltpu.VMEM((2 * L, C), jnp.float32),
            pltpu.SemaphoreType.DMA((2,)),
        ],
    )(_sc_encode_body)


BANK = NROW * 2 * C
WBANK = NROW * C


def _sc_encode_body(tab, xT, dense_hbm, out, dense, xv, idxb, wbuf,
                    rowsb, outv, sems):
    cid = lax.axis_index("c")
    sid = lax.axis_index("s")
    wid = sid * NC + cid

    pltpu.sync_copy(dense_hbm.at[pl.ds(0, 2 * DENSE_ROWS)], dense)

    def loadx(ch, bk):
        base = wid * SQPW + ch * C
        xo = bk * (2 * C)
        pltpu.sync_copy(xT.at[0, pl.ds(base, C)], xv.at[pl.ds(xo, C)])
        pltpu.sync_copy(xT.at[1, pl.ds(base, C)], xv.at[pl.ds(xo + C, C)])

    def p1(bk):
        bo = bk * BANK
        wo = bk * WBANK
        xo = bk * (2 * C)

        def body(g, c2):
            sl = pl.ds(g * 16, 16)
            xs = xv[pl.ds(xo + g * 16, 16)]
            ys = xv[pl.ds(xo + C + g * 16, 16)]
            for si, l in enumerate(STREAM_LEVELS):
                res = float(RES[l])
                px = xs * res
                py = ys * res
                x0 = px.astype(jnp.int32)
                y0 = py.astype(jnp.int32)
                fx = px - x0.astype(jnp.float32)
                fy = py - y0.astype(jnp.float32)
                gx = 1.0 - fx
                gy = 1.0 - fy
                m0 = y0 * jnp.int32(PRIME_I32)
                m1 = m0 + jnp.int32(PRIME_I32)
                x1 = x0 + 1
                lb2 = jnp.int32(2 * l * T)
                mk = jnp.int32(MASK)
                for c4, (hx, hm, w) in enumerate((
                        (x0, m0, gx * gy), (x0, m1, gx * fy),
                        (x1, m0, fx * gy), (x1, m1, fx * fy))):
                    r = si * 4 + c4
                    h3 = (hx ^ hm) & mk
                    hh = ((h3 + h3) - (h3 & 127)) | lb2
                    o = r * 2 * C
                    idxb[pl.ds(bo + o + g * 16, 16)] = hh
                    idxb[pl.ds(bo + o + C + g * 16, 16)] = hh + 128
                    wbuf[pl.ds(wo + r * C + g * 16, 16)] = w
            return c2

        lax.fori_loop(0, NG, body, 0)

    def fire(bk):
        bo = bk * BANK
        for k in range(NROW * 2):
            pltpu.async_copy(
                tab.at[idxb.at[pl.ds(bo + k * C, C)]],
                rowsb.at[pl.ds(bo + k * C, C)], sems.at[bk])

    def drain(bk):
        bo = bk * BANK
        pltpu.make_async_copy(
            tab.at[pl.ds(0, BANK)],
            rowsb.at[pl.ds(bo, BANK)], sems.at[bk]).wait()

    def p2(bk):
        xo = bk * (2 * C)

        def body(g, c2):
            sl = pl.ds(g * 16, 16)
            xs = xv[pl.ds(xo + g * 16, 16)]
            ys = xv[pl.ds(xo + C + g * 16, 16)]
            for l in DENSE_LEVELS:
                res = float(RES[l])
                s = RES[l] + 1
                px = xs * res
                py = ys * res
                x0 = px.astype(jnp.int32)
                y0 = py.astype(jnp.int32)
                fx = px - x0.astype(jnp.float32)
                fy = py - y0.astype(jnp.float32)
                gx = 1.0 - fx
                gy = 1.0 - fy
                d00 = x0 * jnp.int32(2 * s) + (y0 + y0) + jnp.int32(2 * _doff[l])
                d01 = d00 + 2
                d10 = d00 + jnp.int32(2 * s)
                d11 = d10 + 2
                a0 = jnp.zeros((16,), jnp.float32)
                a1 = jnp.zeros((16,), jnp.float32)
                for d, w in ((d00, gx * gy), (d01, gx * fy),
                             (d10, fx * gy), (d11, fx * fy)):
                    a0 = a0 + plsc.load_gather(dense, [d]) * w
                    a1 = a1 + plsc.load_gather(dense, [d + 1]) * w
                outv[2 * l, sl] = a0
                outv[2 * l + 1, sl] = a1
            return c2

        lax.fori_loop(0, NG, body, 0)

    def p3(bk):
        bo = bk * BANK
        wo = bk * WBANK

        def body(g, c2):
            sl = pl.ds(g * 16, 16)
            for si, l in enumerate(STREAM_LEVELS):
                a0 = jnp.zeros((16,), jnp.float32)
                a1 = jnp.zeros((16,), jnp.float32)
                for c4 in range(4):
                    r = si * 4 + c4
                    o = r * 2 * C
                    w = wbuf[pl.ds(wo + r * C + g * 16, 16)]
                    a0 = a0 + rowsb[pl.ds(bo + o + g * 16, 16)] * w
                    a1 = a1 + rowsb[pl.ds(bo + o + C + g * 16, 16)] * w
                outv[2 * l, sl] = a0
                outv[2 * l + 1, sl] = a1
            return c2

        lax.fori_loop(0, NG, body, 0)

    loadx(0, 0)
    p1(0)
    fire(0)

    def chunk(ch, carry):
        par = lax.rem(ch, 2)
        npar = 1 - par

        @pl.when(ch + 1 < NCHUNK_S)
        def _():
            loadx(ch + 1, npar)
            p1(npar)
            fire(npar)

        p2(par)
        drain(par)
        p3(par)
        pltpu.sync_copy(outv, out.at[:, pl.ds(wid * SQPW + ch * C, C)])
        return carry

    lax.fori_loop(0, NCHUNK_S, chunk, 0)


BN = 512


def _mlp_body(e_ref, w1t_ref, b1_ref, w2t_ref, b2_ref, w3_ref, b3_ref,
              o_ref):
    e = e_ref[...]
    h1 = jax.lax.dot_general(w1t_ref[...], e, (((1,), (0,)), ((), ())),
                             preferred_element_type=jnp.float32)
    h1 = jnp.maximum(h1 + b1_ref[...], 0.0)
    h2 = jax.lax.dot_general(w2t_ref[...], h1, (((1,), (0,)), ((), ())),
                             preferred_element_type=jnp.float32)
    h2 = jnp.maximum(h2 + b2_ref[...], 0.0)
    o_ref[...] = jnp.sum(h2 * w3_ref[...], axis=0, keepdims=True) + b3_ref[...]


def _mlp(enc, w1t, b1c, w2t, b2c, w3, b3c):
    return pl.pallas_call(
        _mlp_body,
        grid=(SN // BN,),
        in_specs=[
            pl.BlockSpec((2 * L, BN), lambda i: (0, i)),
            pl.BlockSpec((128, 2 * L), lambda i: (0, 0)),
            pl.BlockSpec((128, 1), lambda i: (0, 0)),
            pl.BlockSpec((128, 128), lambda i: (0, 0)),
            pl.BlockSpec((128, 1), lambda i: (0, 0)),
            pl.BlockSpec((128, 1), lambda i: (0, 0)),
            pl.BlockSpec((1, 1), lambda i: (0, 0)),
        ],
        out_specs=pl.BlockSpec((1, BN), lambda i: (0, i)),
        out_shape=jax.ShapeDtypeStruct((1, SN), jnp.float32),
    )(enc, w1t, b1c, w2t, b2c, w3, b3c)


def kernel(x, tables, W1, b1, W2, b2, W3, b3):
    xT = x.T
    tab = tables.reshape(L, T // 128, 128, F).transpose(0, 1, 3, 2)
    tab = tab.reshape(L * T * F)
    dense_arr = _make_sc_stage()(tab, jnp.asarray(STAGE_IDX))
    encode = _make_sc_encode()
    w1t, b1c = W1.T, b1.reshape(128, 1)
    w2t, b2c = W2.T, b2.reshape(128, 1)
    b3c = b3.reshape(1, 1)
    outs = []
    for s in range(SLABS):
        enc = encode(tab, xT[:, s * SN:(s + 1) * SN], dense_arr)
        outs.append(_mlp(enc, w1t, b1c, w2t, b2c, W3, b3c))
    return jnp.concatenate(outs, axis=1).reshape(N, 1)
